# baseline (speedup 1.0000x reference)
import functools

import numpy as np
import jax
import jax.numpy as jnp
from jax import lax
from jax.experimental import pallas as pl
from jax.experimental.pallas import tpu as pltpu
from jax.experimental.pallas import tpu_sc as plsc

VOCAB = 1000000
HIDDEN = 64
LABELS = 50
BATCH = 4096
SEQ = 200

HALF2 = 1 << 19
BLK = 2048
NCHUNK = HALF2 // BLK
FULL_CHUNKS = (VOCAB - HALF2) // BLK
TAILW = VOCAB - HALF2 - FULL_CHUNKS * BLK
TAIL_DMA = (TAILW // 128) * 128
TAIL_VEC = TAILW - TAIL_DMA
NC = 2
NS = 16
NW = NC * NS
ROWS_PER_W = BATCH // NW
CHUNK = 100
CHUNKS_PER_ROW = SEQ // CHUNK
CHUNKS_PER_W = ROWS_PER_W * CHUNKS_PER_ROW
NVEC = HIDDEN // 16
PAIRW = 2 * HIDDEN
NBUF = 2


LAST_E2_BLOCK = (VOCAB - BLK) // BLK

_SIGMA = np.empty(HIDDEN, dtype=np.int64)
for _p in range(16):
    _SIGMA[_p] = 2 * _p
    _SIGMA[16 + _p] = 2 * _p + 1
    _SIGMA[32 + _p] = 32 + 2 * _p
    _SIGMA[48 + _p] = 33 + 2 * _p
_INV_SIGMA = np.empty(HIDDEN, dtype=np.int64)
_INV_SIGMA[_SIGMA] = np.arange(HIDDEN)


def _proj_body(e1_ref, e2_ref, e2_hbm, tail_ref, w_ref, b_ref, o_ref,
               e2_v, sem):
    i = pl.program_id(0)

    @pl.when(i == FULL_CHUNKS)
    def _():
        cp = pltpu.make_async_copy(
            e2_hbm.at[:, pl.ds(HALF2 + FULL_CHUNKS * BLK, TAIL_DMA)],
            e2_v.at[:, pl.ds(0, TAIL_DMA)], sem)
        cp.start()
        cp.wait()
        e2_v[:, pl.ds(TAIL_DMA, TAIL_VEC)] = tail_ref[...]

    e2 = jnp.where(i == FULL_CHUNKS, e2_v[...], e2_ref[...])
    dn = (((0,), (1,)), ((), ()))
    wb = w_ref[...].astype(jnp.bfloat16)
    t1 = lax.dot_general(e1_ref[...].astype(jnp.bfloat16), wb, dn,
                         preferred_element_type=jnp.float32)
    t2 = lax.dot_general(e2.astype(jnp.bfloat16), wb, dn,
                         preferred_element_type=jnp.float32)
    o_ref[...] = jnp.concatenate(
        [t1 + b_ref[...], t2 + b_ref[...]], axis=1).astype(jnp.bfloat16)


def _project(embt, tail_e, Wp, bp):
    return pl.pallas_call(
        _proj_body,
        out_shape=jax.ShapeDtypeStruct((HALF2, PAIRW), jnp.bfloat16),
        grid=(NCHUNK,),
        in_specs=[
            pl.BlockSpec((HIDDEN, BLK), lambda i: (0, i)),
            pl.BlockSpec(
                (HIDDEN, BLK),
                lambda i: (0, jnp.minimum(i + NCHUNK, LAST_E2_BLOCK))),
            pl.BlockSpec(memory_space=pltpu.MemorySpace.HBM),
            pl.BlockSpec((HIDDEN, TAIL_VEC), lambda i: (0, 0)),
            pl.BlockSpec((HIDDEN, HIDDEN), lambda i: (0, 0)),
            pl.BlockSpec((1, HIDDEN), lambda i: (0, 0)),
        ],
        out_specs=pl.BlockSpec((BLK, PAIRW), lambda i: (i, 0)),
        scratch_shapes=[
            pltpu.VMEM((HIDDEN, BLK), jnp.float32),
            pltpu.SemaphoreType.DMA,
        ],
    )(embt, embt, embt, tail_e, Wp, bp)


def _pool_body(kv_hbm, off_hbm, tab_hbm, h_hbm, kv_v, off_v, rows_v, h_v,
               *sems):
    wid = lax.axis_index("s") * NC + lax.axis_index("c")

    pltpu.sync_copy(kv_hbm.at[pl.ds(wid * CHUNKS_PER_W, CHUNKS_PER_W)], kv_v)
    pltpu.sync_copy(off_hbm.at[pl.ds(wid * ROWS_PER_W, ROWS_PER_W)], off_v)

    inv = jnp.float32(1.0 / SEQ)

    def fire(r, b):
        c0 = r * CHUNKS_PER_ROW
        pltpu.async_copy(
            tab_hbm.at[kv_v.at[c0]], rows_v.at[b].at[pl.ds(0, CHUNK)],
            sems[b])
        pltpu.async_copy(
            tab_hbm.at[kv_v.at[c0 + 1]], rows_v.at[b].at[pl.ds(CHUNK, CHUNK)],
            sems[b])

    def drain(b):
        pltpu.make_async_copy(
            tab_hbm.at[kv_v.at[0]], rows_v.at[b].at[pl.ds(0, CHUNK)],
            sems[b]).wait()
        pltpu.make_async_copy(
            tab_hbm.at[kv_v.at[0]], rows_v.at[b].at[pl.ds(CHUNK, CHUNK)],
            sems[b]).wait()

    for b in range(NBUF):
        fire(b, b)

    @pl.loop(0, ROWS_PER_W, step=NBUF)
    def _outer(r0):
        for b in range(NBUF):
            r = r0 + b
            drain(b)

            def acc_group(t, base, nu, lane0, acc):
                off_vec = off_v[r, pl.ds(base, 16)]
                for u in range(nu):
                    off = off_vec[lane0 + u]
                    j = t * 16 + u
                    lo = rows_v[b, j, pl.ds(off, 32)]
                    hi = rows_v[b, j, pl.ds(off + 32, 32)]
                    a0, b0 = plsc.unpack(
                        lo, format=plsc.PackFormat.INTERLEAVED,
                        preferred_element_type=jnp.float32)
                    a1, b1 = plsc.unpack(
                        hi, format=plsc.PackFormat.INTERLEAVED,
                        preferred_element_type=jnp.float32)
                    acc = (acc[0] + a0, acc[1] + b0, acc[2] + a1, acc[3] + b1)
                return acc

            acc = lax.fori_loop(
                0, SEQ // 16, lambda t, a: acc_group(t, t * 16, 16, 0, a),
                tuple(jnp.zeros((16,), jnp.float32) for _ in range(NVEC)))
            acc = acc_group(SEQ // 16, SEQ - 16, SEQ % 16, 16 - SEQ % 16, acc)
            for d in range(NVEC):
                h_v[r, pl.ds(16 * d, 16)] = acc[d] * inv

            nxt = r + NBUF

            @pl.when(nxt < ROWS_PER_W)
            def _():
                fire(nxt, b)

    pltpu.sync_copy(h_v, h_hbm.at[pl.ds(wid * ROWS_PER_W, ROWS_PER_W)])


_pool = functools.partial(
    pl.kernel,
    mesh=plsc.VectorSubcoreMesh(core_axis_name="c", subcore_axis_name="s"),
    out_type=jax.ShapeDtypeStruct((BATCH, HIDDEN), jnp.float32),
    scratch_types=[
        pltpu.VMEM((CHUNKS_PER_W, CHUNK), jnp.int32),
        pltpu.VMEM((ROWS_PER_W, SEQ), jnp.int32),
        pltpu.VMEM((NBUF, SEQ, PAIRW), jnp.bfloat16),
        pltpu.VMEM((ROWS_PER_W, HIDDEN), jnp.float32),
    ] + [pltpu.SemaphoreType.DMA] * NBUF,
    compiler_params=pltpu.CompilerParams(use_tc_tiling_on_sc=False,
                                         needs_layout_passes=False),
)(_pool_body)


@jax.jit
def kernel(x, emb, W, b):
    xi = x.astype(jnp.int32)
    Wfull = jnp.zeros((HIDDEN, HIDDEN), jnp.float32).at[:LABELS].set(W)
    bfull = jnp.zeros((HIDDEN,), jnp.float32).at[:LABELS].set(b)
    Wp = Wfull[_INV_SIGMA]
    bp = bfull[_INV_SIGMA].reshape(1, HIDDEN)
    embt = emb.T
    tail_e = lax.slice(embt, (0, VOCAB - TAIL_VEC), (HIDDEN, VOCAB))
    tab = _project(embt, tail_e, Wp, bp)
    kv = (xi & (HALF2 - 1)).reshape(BATCH * CHUNKS_PER_ROW, CHUNK)
    off = (((xi >> 19) & 1) * HIDDEN).reshape(BATCH, SEQ)
    h = _pool(kv, off, tab)
    return h[:, :LABELS]


# speedup vs baseline: 2.3478x; 2.3478x over previous
import functools

import numpy as np
import jax
import jax.numpy as jnp
from jax import lax
from jax.experimental import pallas as pl
from jax.experimental.pallas import tpu as pltpu
from jax.experimental.pallas import tpu_sc as plsc

VOCAB = 1000000
HIDDEN = 64
LABELS = 50
BATCH = 4096
SEQ = 200

QV = 1 << 18
ENT32 = HIDDEN // 2
BLK = 2048
NCHUNK = QV // BLK
FULL3 = (VOCAB - 3 * QV) // BLK
TAILW = VOCAB - 3 * QV - FULL3 * BLK
TAIL_DMA = (TAILW // 128) * 128
TAIL_VEC = TAILW - TAIL_DMA
LAST_BLOCK = (VOCAB - BLK) // BLK

NC = 2
NS = 16
NW = NC * NS
ROWS_PER_W = BATCH // NW
CHUNK = 100
CHUNKS_PER_ROW = SEQ // CHUNK
CHUNKS_PER_W = ROWS_PER_W * CHUNKS_PER_ROW
NVEC = HIDDEN // 16
NBUF = 4

_LO_IDX = np.r_[0:16, 32:48]
_HI_IDX = np.r_[16:32, 48:64]


def _proj_body(e0_ref, e1_ref, e2_ref, e3_ref, e3_hbm, tail_ref,
               wlo_ref, whi_ref, blo_ref, bhi_ref, o_ref, e3_v, sem):
    i = pl.program_id(0)

    @pl.when(i == FULL3)
    def _():
        cp = pltpu.make_async_copy(
            e3_hbm.at[:, pl.ds(3 * QV + FULL3 * BLK, TAIL_DMA)],
            e3_v.at[:, pl.ds(0, TAIL_DMA)], sem)
        cp.start()
        cp.wait()
        e3_v[:, pl.ds(TAIL_DMA, TAIL_VEC)] = tail_ref[...]

    dn = (((0,), (1,)), ((), ()))
    wlo = wlo_ref[...].astype(jnp.bfloat16)
    whi = whi_ref[...].astype(jnp.bfloat16)

    def pack(e):
        eb = e.astype(jnp.bfloat16)
        tl = lax.dot_general(eb, wlo, dn,
                             preferred_element_type=jnp.float32) + blo_ref[...]
        th = lax.dot_general(eb, whi, dn,
                             preferred_element_type=jnp.float32) + bhi_ref[...]
        ul = lax.bitcast_convert_type(
            tl.astype(jnp.bfloat16), jnp.uint16).astype(jnp.uint32)
        uh = lax.bitcast_convert_type(
            th.astype(jnp.bfloat16), jnp.uint16).astype(jnp.uint32)
        return (ul | (uh << 16)).astype(jnp.int32)

    e3 = jnp.where(i == FULL3, e3_v[...], e3_ref[...])
    o_ref[...] = jnp.concatenate(
        [pack(e0_ref[...]), pack(e1_ref[...]), pack(e2_ref[...]), pack(e3)],
        axis=1)


def _project(embt, tail_e, Wlo, Whi, blo, bhi):
    nb = NCHUNK
    return pl.pallas_call(
        _proj_body,
        out_shape=jax.ShapeDtypeStruct((QV, 4 * ENT32), jnp.int32),
        grid=(NCHUNK,),
        in_specs=[
            pl.BlockSpec((HIDDEN, BLK), lambda i: (0, i)),
            pl.BlockSpec((HIDDEN, BLK), lambda i: (0, i + nb)),
            pl.BlockSpec((HIDDEN, BLK), lambda i: (0, i + 2 * nb)),
            pl.BlockSpec(
                (HIDDEN, BLK),
                lambda i: (0, jnp.minimum(i + 3 * nb, LAST_BLOCK))),
            pl.BlockSpec(memory_space=pltpu.MemorySpace.HBM),
            pl.BlockSpec((HIDDEN, TAIL_VEC), lambda i: (0, 0)),
            pl.BlockSpec((ENT32, HIDDEN), lambda i: (0, 0)),
            pl.BlockSpec((ENT32, HIDDEN), lambda i: (0, 0)),
            pl.BlockSpec((1, ENT32), lambda i: (0, 0)),
            pl.BlockSpec((1, ENT32), lambda i: (0, 0)),
        ],
        out_specs=pl.BlockSpec((BLK, 4 * ENT32), lambda i: (i, 0)),
        scratch_shapes=[
            pltpu.VMEM((HIDDEN, BLK), jnp.float32),
            pltpu.SemaphoreType.DMA,
        ],
    )(embt, embt, embt, embt, embt, tail_e, Wlo, Whi, blo, bhi)


def _pool_body(kv_hbm, tab_hbm, h_hbm, kv_v, rows_v, h_v, *sems):
    wid = lax.axis_index("s") * NC + lax.axis_index("c")

    pltpu.sync_copy(kv_hbm.at[pl.ds(wid * CHUNKS_PER_W, CHUNKS_PER_W)], kv_v)

    inv = jnp.float32(1.0 / SEQ)

    def fire(r, b):
        c0 = r * CHUNKS_PER_ROW
        pltpu.async_copy(
            tab_hbm.at[kv_v.at[c0]], rows_v.at[b].at[pl.ds(0, CHUNK)],
            sems[b])
        pltpu.async_copy(
            tab_hbm.at[kv_v.at[c0 + 1]], rows_v.at[b].at[pl.ds(CHUNK, CHUNK)],
            sems[b])

    def drain(b):
        pltpu.make_async_copy(
            tab_hbm.at[kv_v.at[0]], rows_v.at[b].at[pl.ds(0, CHUNK)],
            sems[b]).wait()
        pltpu.make_async_copy(
            tab_hbm.at[kv_v.at[0]], rows_v.at[b].at[pl.ds(CHUNK, CHUNK)],
            sems[b]).wait()

    for b in range(NBUF):
        fire(b, b)

    @pl.loop(0, ROWS_PER_W, step=NBUF)
    def _outer(r0):
        for b in range(NBUF):
            r = r0 + b
            drain(b)

            def acc_body(j, acc):
                w0 = rows_v[b, j, pl.ds(0, 16)]
                w1 = rows_v[b, j, pl.ds(16, 16)]
                a0, b0 = plsc.unpack(
                    plsc.bitcast(w0, jnp.bfloat16),
                    format=plsc.PackFormat.INTERLEAVED,
                    preferred_element_type=jnp.float32)
                a1, b1 = plsc.unpack(
                    plsc.bitcast(w1, jnp.bfloat16),
                    format=plsc.PackFormat.INTERLEAVED,
                    preferred_element_type=jnp.float32)
                return (acc[0] + a0, acc[1] + b0, acc[2] + a1, acc[3] + b1)

            acc = lax.fori_loop(
                0, SEQ, acc_body,
                tuple(jnp.zeros((16,), jnp.float32) for _ in range(NVEC)),
                unroll=8)
            for d in range(NVEC):
                h_v[r, pl.ds(16 * d, 16)] = acc[d] * inv

            nxt = r + NBUF

            @pl.when(nxt < ROWS_PER_W)
            def _():
                fire(nxt, b)

    pltpu.sync_copy(h_v, h_hbm.at[pl.ds(wid * ROWS_PER_W, ROWS_PER_W)])


_pool = functools.partial(
    pl.kernel,
    mesh=plsc.VectorSubcoreMesh(core_axis_name="c", subcore_axis_name="s"),
    out_type=jax.ShapeDtypeStruct((BATCH, HIDDEN), jnp.float32),
    scratch_types=[
        pltpu.VMEM((CHUNKS_PER_W, CHUNK), jnp.int32),
        pltpu.VMEM((NBUF, SEQ, ENT32), jnp.int32),
        pltpu.VMEM((ROWS_PER_W, HIDDEN), jnp.float32),
    ] + [pltpu.SemaphoreType.DMA] * NBUF,
    compiler_params=pltpu.CompilerParams(use_tc_tiling_on_sc=False,
                                         needs_layout_passes=False),
)(_pool_body)


@jax.jit
def kernel(x, emb, W, b):
    xi = x.astype(jnp.int32)
    Wfull = jnp.zeros((HIDDEN, HIDDEN), jnp.float32).at[:LABELS].set(W)
    bfull = jnp.zeros((HIDDEN,), jnp.float32).at[:LABELS].set(b)
    Wlo = Wfull[_LO_IDX]
    Whi = Wfull[_HI_IDX]
    blo = bfull[_LO_IDX].reshape(1, ENT32)
    bhi = bfull[_HI_IDX].reshape(1, ENT32)
    embt = emb.T
    tail_e = lax.slice(embt, (0, VOCAB - TAIL_VEC), (HIDDEN, VOCAB))
    tab = _project(embt, tail_e, Wlo, Whi, blo, bhi)
    tabv = tab.reshape(4 * QV, ENT32)
    kv = (4 * (xi & (QV - 1)) + (xi >> 18)).reshape(
        BATCH * CHUNKS_PER_ROW, CHUNK)
    h = _pool(kv, tabv)
    return h[:, :LABELS]


# speedup vs baseline: 2.4997x; 1.0647x over previous
import functools

import numpy as np
import jax
import jax.numpy as jnp
from jax import lax
from jax.experimental import pallas as pl
from jax.experimental.pallas import tpu as pltpu
from jax.experimental.pallas import tpu_sc as plsc

VOCAB = 1000000
HIDDEN = 64
LABELS = 50
BATCH = 4096
SEQ = 200

QV = 1 << 18
ENT32 = HIDDEN // 2
BLK = 8192
NCHUNK = QV // BLK
FULL3 = (VOCAB - 3 * QV) // BLK
TAILW = VOCAB - 3 * QV - FULL3 * BLK
TAIL_DMA = (TAILW // 128) * 128
TAIL_VEC = TAILW - TAIL_DMA
LAST_BLOCK = (VOCAB - BLK) // BLK

NC = 2
NS = 16
NW = NC * NS
ROWS_PER_W = BATCH // NW
CHUNK = 100
CHUNKS_PER_ROW = SEQ // CHUNK
CHUNKS_PER_W = ROWS_PER_W * CHUNKS_PER_ROW
NVEC = HIDDEN // 16
NBUF = 4

_LO_IDX = np.r_[0:16, 32:48]
_HI_IDX = np.r_[16:32, 48:64]


def _proj_body(e0_ref, e1_ref, e2_ref, e3_ref, e3_hbm, tail_ref,
               wlo_ref, whi_ref, blo_ref, bhi_ref, o_ref, e3_v, sem):
    i = pl.program_id(0)

    @pl.when(i == FULL3)
    def _():
        cp = pltpu.make_async_copy(
            e3_hbm.at[:, pl.ds(3 * QV + FULL3 * BLK, TAIL_DMA)],
            e3_v.at[:, pl.ds(0, TAIL_DMA)], sem)
        cp.start()
        cp.wait()
        e3_v[:, pl.ds(TAIL_DMA, TAIL_VEC)] = tail_ref[...]

    dn = (((0,), (1,)), ((), ()))
    wlo = wlo_ref[...].astype(jnp.bfloat16)
    whi = whi_ref[...].astype(jnp.bfloat16)

    def pack(e):
        eb = e.astype(jnp.bfloat16)
        tl = lax.dot_general(eb, wlo, dn,
                             preferred_element_type=jnp.float32) + blo_ref[...]
        th = lax.dot_general(eb, whi, dn,
                             preferred_element_type=jnp.float32) + bhi_ref[...]
        ul = lax.bitcast_convert_type(
            tl.astype(jnp.bfloat16), jnp.uint16).astype(jnp.uint32)
        uh = lax.bitcast_convert_type(
            th.astype(jnp.bfloat16), jnp.uint16).astype(jnp.uint32)
        return (ul | (uh << 16)).astype(jnp.int32)

    e3 = jnp.where(i == FULL3, e3_v[...], e3_ref[...])
    o_ref[...] = jnp.concatenate(
        [pack(e0_ref[...]), pack(e1_ref[...]), pack(e2_ref[...]), pack(e3)],
        axis=1)


def _project(embt, tail_e, Wlo, Whi, blo, bhi):
    nb = NCHUNK
    return pl.pallas_call(
        _proj_body,
        out_shape=jax.ShapeDtypeStruct((QV, 4 * ENT32), jnp.int32),
        grid=(NCHUNK,),
        in_specs=[
            pl.BlockSpec((HIDDEN, BLK), lambda i: (0, i)),
            pl.BlockSpec((HIDDEN, BLK), lambda i: (0, i + nb)),
            pl.BlockSpec((HIDDEN, BLK), lambda i: (0, i + 2 * nb)),
            pl.BlockSpec(
                (HIDDEN, BLK),
                lambda i: (0, jnp.minimum(i + 3 * nb, LAST_BLOCK))),
            pl.BlockSpec(memory_space=pltpu.MemorySpace.HBM),
            pl.BlockSpec((HIDDEN, TAIL_VEC), lambda i: (0, 0)),
            pl.BlockSpec((ENT32, HIDDEN), lambda i: (0, 0)),
            pl.BlockSpec((ENT32, HIDDEN), lambda i: (0, 0)),
            pl.BlockSpec((1, ENT32), lambda i: (0, 0)),
            pl.BlockSpec((1, ENT32), lambda i: (0, 0)),
        ],
        out_specs=pl.BlockSpec((BLK, 4 * ENT32), lambda i: (i, 0)),
        compiler_params=pltpu.CompilerParams(
            fuse_transposed_lhs_in_matmul=True),
        scratch_shapes=[
            pltpu.VMEM((HIDDEN, BLK), jnp.float32),
            pltpu.SemaphoreType.DMA,
        ],
    )(embt, embt, embt, embt, embt, tail_e, Wlo, Whi, blo, bhi)


def _pool_body(kv_hbm, tab_hbm, h_hbm, kv_v, rows_v, h_v, *sems):
    wid = lax.axis_index("s") * NC + lax.axis_index("c")

    pltpu.sync_copy(kv_hbm.at[pl.ds(wid * CHUNKS_PER_W, CHUNKS_PER_W)], kv_v)

    inv = jnp.float32(1.0 / SEQ)

    def fire(r, b):
        c0 = r * CHUNKS_PER_ROW
        pltpu.async_copy(
            tab_hbm.at[kv_v.at[c0]], rows_v.at[b].at[pl.ds(0, CHUNK)],
            sems[b])
        pltpu.async_copy(
            tab_hbm.at[kv_v.at[c0 + 1]], rows_v.at[b].at[pl.ds(CHUNK, CHUNK)],
            sems[b])

    def drain(b):
        pltpu.make_async_copy(
            tab_hbm.at[kv_v.at[0]], rows_v.at[b].at[pl.ds(0, CHUNK)],
            sems[b]).wait()
        pltpu.make_async_copy(
            tab_hbm.at[kv_v.at[0]], rows_v.at[b].at[pl.ds(CHUNK, CHUNK)],
            sems[b]).wait()

    for b in range(NBUF):
        fire(b, b)

    @pl.loop(0, ROWS_PER_W, step=NBUF)
    def _outer(r0):
        for b in range(NBUF):
            r = r0 + b
            drain(b)

            def acc_body(j, acc):
                w0 = rows_v[b, j, pl.ds(0, 16)]
                w1 = rows_v[b, j, pl.ds(16, 16)]
                a0, b0 = plsc.unpack(
                    plsc.bitcast(w0, jnp.bfloat16),
                    format=plsc.PackFormat.INTERLEAVED,
                    preferred_element_type=jnp.float32)
                a1, b1 = plsc.unpack(
                    plsc.bitcast(w1, jnp.bfloat16),
                    format=plsc.PackFormat.INTERLEAVED,
                    preferred_element_type=jnp.float32)
                return (acc[0] + a0, acc[1] + b0, acc[2] + a1, acc[3] + b1)

            acc = lax.fori_loop(
                0, SEQ, acc_body,
                tuple(jnp.zeros((16,), jnp.float32) for _ in range(NVEC)),
                unroll=8)
            for d in range(NVEC):
                h_v[r, pl.ds(16 * d, 16)] = acc[d] * inv

            nxt = r + NBUF

            @pl.when(nxt < ROWS_PER_W)
            def _():
                fire(nxt, b)

    pltpu.sync_copy(h_v, h_hbm.at[pl.ds(wid * ROWS_PER_W, ROWS_PER_W)])


_pool = functools.partial(
    pl.kernel,
    mesh=plsc.VectorSubcoreMesh(core_axis_name="c", subcore_axis_name="s"),
    out_type=jax.ShapeDtypeStruct((BATCH, HIDDEN), jnp.float32),
    scratch_types=[
        pltpu.VMEM((CHUNKS_PER_W, CHUNK), jnp.int32),
        pltpu.VMEM((NBUF, SEQ, ENT32), jnp.int32),
        pltpu.VMEM((ROWS_PER_W, HIDDEN), jnp.float32),
    ] + [pltpu.SemaphoreType.DMA] * NBUF,
    compiler_params=pltpu.CompilerParams(use_tc_tiling_on_sc=False,
                                         needs_layout_passes=False),
)(_pool_body)


@jax.jit
def kernel(x, emb, W, b):
    xi = x.astype(jnp.int32)
    Wfull = jnp.zeros((HIDDEN, HIDDEN), jnp.float32).at[:LABELS].set(W)
    bfull = jnp.zeros((HIDDEN,), jnp.float32).at[:LABELS].set(b)
    Wlo = Wfull[_LO_IDX]
    Whi = Wfull[_HI_IDX]
    blo = bfull[_LO_IDX].reshape(1, ENT32)
    bhi = bfull[_HI_IDX].reshape(1, ENT32)
    embt = emb.T
    tail_e = lax.slice(embt, (0, VOCAB - TAIL_VEC), (HIDDEN, VOCAB))
    tab = _project(embt, tail_e, Wlo, Whi, blo, bhi)
    tabv = tab.reshape(4 * QV, ENT32)
    kv = (4 * (xi & (QV - 1)) + (xi >> 18)).reshape(
        BATCH * CHUNKS_PER_ROW, CHUNK)
    h = _pool(kv, tabv)
    return h[:, :LABELS]
